# baseline (speedup 1.0000x reference)
import functools

import jax
import jax.numpy as jnp
from jax import lax
from jax.experimental import pallas as pl
from jax.experimental.pallas import tpu as pltpu
from jax.experimental.pallas import tpu_sc as plsc

N_PTS = 50000
K = 16
INC = 512
HIDDEN = INC // 4

NC = 2
NS = 16
NW = NC * NS

G = 8
NB = 8

PAD_N = 50176
HALF = PAD_N // 2
PPW0H = 832
PPW1H = 736
assert (PPW0H + PPW1H) * NS == HALF
OFF1H = PPW0H * NS
NGH0H = PPW0H // (2 * G)
NGH1H = PPW1H // (2 * G)
PAD_N_IDX = PAD_N + PPW0H - PPW1H

_MESH = plsc.VectorSubcoreMesh(core_axis_name="c", subcore_axis_name="s")


def _make_gather(hoff):
    @functools.partial(
        pl.kernel,
        mesh=_MESH,
        out_type=jax.ShapeDtypeStruct((HALF, INC), jnp.float32),
        scratch_types=[
            pltpu.VMEM((PPW0H * K,), jnp.int32),
            pltpu.VMEM((NB, K, INC), jnp.float32),
            pltpu.VMEM((2, G, INC), jnp.float32),
            pltpu.SemaphoreType.DMA,
            pltpu.SemaphoreType.DMA,
        ],
    )
    def _gather_sum(f_hbm, idx_hbm, sum_hbm, idx_v, rows_v, out_v, gsem,
                    osem):
        c = lax.axis_index("c")
        s = lax.axis_index("s")
        on0 = c == 0
        base = lax.select(on0, s * PPW0H, OFF1H + s * PPW1H)
        ppw = lax.select(on0, jnp.int32(PPW0H), jnp.int32(PPW1H))
        ng_half = lax.select(on0, jnp.int32(NGH0H), jnp.int32(NGH1H))

        pltpu.sync_copy(idx_hbm.at[pl.ds((hoff + base) * K, PPW0H * K)],
                        idx_v)

        def fire(p, b):
            ivec = idx_v[pl.ds(p * K, K)]
            pltpu.async_copy(f_hbm.at[ivec], rows_v.at[b], gsem)

        def wait_gather(b):
            pltpu.make_async_copy(f_hbm.at[pl.ds(0, K)], rows_v.at[b],
                                  gsem).wait()

        def wait_out(ob):
            pltpu.make_async_copy(out_v.at[ob], sum_hbm.at[pl.ds(0, G)],
                                  osem).wait()

        for b in range(NB):
            fire(b, b)

        def reduce_point(b, ob, j):
            def cbody(cc, carry):
                col = cc * 16
                vals = [rows_v[b, k, pl.ds(col, 16)] for k in range(K)]
                while len(vals) > 1:
                    vals = [vals[i] + vals[i + 1]
                            for i in range(0, len(vals), 2)]
                out_v[ob, j, pl.ds(col, 16)] = vals[0]
                return carry

            lax.fori_loop(0, INC // 16, cbody, 0)

        def pair_body(m, carry):
            @pl.when(m < ng_half)
            def _():
                for ob in range(2):
                    g = m * 2 + ob

                    @pl.when(m > 0)
                    def _():
                        wait_out(ob)

                    for j in range(G):
                        b = j % NB
                        p = g * G + j
                        wait_gather(b)
                        reduce_point(b, ob, j)
                        nxt = p + NB

                        @pl.when(nxt < ppw)
                        def _():
                            fire(nxt, b)

                    pltpu.async_copy(out_v.at[ob],
                                     sum_hbm.at[pl.ds(base + g * G, G)],
                                     osem)
            return carry

        lax.fori_loop(0, NGH0H, pair_body, 0)

        wait_out(0)
        wait_out(1)

    return _gather_sum


_GATHER_LO = _make_gather(0)
_GATHER_HI = _make_gather(HALF)

_BR = 784
_NBLK = HALF // _BR


def _mlp_body(f_ref, s_ref, w1_ref, b1_ref, w2_ref, b2_ref, o_ref):
    avg = s_ref[...] * (1.0 / K)
    h = jnp.maximum(
        jnp.dot(avg, w1_ref[...], preferred_element_type=jnp.float32)
        + b1_ref[...], 0.0)
    logits = (jnp.dot(h, w2_ref[...], preferred_element_type=jnp.float32)
              + b2_ref[...])
    o_ref[...] = f_ref[...] * jax.nn.sigmoid(logits)


def _mlp_body_aliased(dst_ref, f_ref, s_ref, w1_ref, b1_ref, w2_ref, b2_ref,
                      o_ref):
    del dst_ref
    _mlp_body(f_ref, s_ref, w1_ref, b1_ref, w2_ref, b2_ref, o_ref)


def _mlp_half(F, sums, W1, b1, W2, b2, blk_off, dst=None):
    specs = [
        pl.BlockSpec((_BR, INC), lambda i: (i + blk_off, 0)),
        pl.BlockSpec((_BR, INC), lambda i: (i, 0)),
        pl.BlockSpec((INC, HIDDEN), lambda i: (0, 0)),
        pl.BlockSpec((1, HIDDEN), lambda i: (0, 0)),
        pl.BlockSpec((HIDDEN, INC), lambda i: (0, 0)),
        pl.BlockSpec((1, INC), lambda i: (0, 0)),
    ]
    args = (F, sums, W1, b1, W2, b2)
    body = _mlp_body
    aliases = {}
    if dst is not None:
        specs = [pl.BlockSpec(memory_space=pltpu.MemorySpace.HBM)] + specs
        args = (dst,) + args
        body = _mlp_body_aliased
        aliases = {0: 0}
    return pl.pallas_call(
        body,
        grid=(_NBLK,),
        in_specs=specs,
        out_specs=pl.BlockSpec((_BR, INC), lambda i: (i + blk_off, 0)),
        out_shape=jax.ShapeDtypeStruct((N_PTS, INC), jnp.float32),
        input_output_aliases=aliases,
        compiler_params=pltpu.CompilerParams(
            dimension_semantics=("arbitrary",)),
    )(*args)


def kernel(F, idx, W1, b1, W2, b2):
    idx32 = idx.astype(jnp.int32)
    idx_pad = jnp.pad(idx32, ((0, PAD_N_IDX - N_PTS), (0, 0)))
    idx_flat = idx_pad.reshape(PAD_N_IDX * K)
    sums_lo = _GATHER_LO(F, idx_flat)
    sums_hi = _GATHER_HI(F, idx_flat)
    b1r = b1.reshape(1, HIDDEN)
    b2r = b2.reshape(1, INC)
    out_lo = _mlp_half(F, sums_lo, W1, b1r, W2, b2r, 0)
    return _mlp_half(F, sums_hi, W1, b1r, W2, b2r, _NBLK, dst=out_lo)


# speedup vs baseline: 1.0363x; 1.0363x over previous
import functools

import jax
import jax.numpy as jnp
from jax import lax
from jax.experimental import pallas as pl
from jax.experimental.pallas import tpu as pltpu
from jax.experimental.pallas import tpu_sc as plsc

N_PTS = 50000
K = 16
INC = 512
HIDDEN = INC // 4

NC = 2
NS = 16
NW = NC * NS

G = 8
NB = 8

PPW = -(-N_PTS // (NW * G)) * G
PAD_N = PPW * NW
PPW0 = 1664
PPW1 = 1472
assert PPW0 * NS + PPW1 * NS == PAD_N
OFF1 = PPW0 * NS
NGH0 = PPW0 // (2 * G)
NGH1 = PPW1 // (2 * G)
PAD_N_IDX = PAD_N + PPW0 - PPW1

_MESH = plsc.VectorSubcoreMesh(core_axis_name="c", subcore_axis_name="s")


@functools.partial(
    pl.kernel,
    mesh=_MESH,
    out_type=jax.ShapeDtypeStruct((PAD_N, INC), jnp.float32),
    scratch_types=[
        pltpu.VMEM((PPW0 * K,), jnp.int32),
        pltpu.VMEM((NB, K, INC), jnp.float32),
        pltpu.VMEM((2, G, INC), jnp.float32),
        pltpu.SemaphoreType.DMA,
        pltpu.SemaphoreType.DMA,
    ],
)
def _gather_sum(f_hbm, idx_hbm, sum_hbm, idx_v, rows_v, out_v, gsem, osem):
    c = lax.axis_index("c")
    s = lax.axis_index("s")
    on0 = c == 0
    base = lax.select(on0, s * PPW0, OFF1 + s * PPW1)
    ppw = lax.select(on0, jnp.int32(PPW0), jnp.int32(PPW1))
    ng_half = lax.select(on0, jnp.int32(NGH0), jnp.int32(NGH1))

    pltpu.sync_copy(idx_hbm.at[pl.ds(base * K, PPW0 * K)], idx_v)

    def fire(p, b):
        ivec = idx_v[pl.ds(p * K, K)]
        pltpu.async_copy(f_hbm.at[ivec], rows_v.at[b], gsem)

    def wait_gather(b):
        pltpu.make_async_copy(f_hbm.at[pl.ds(0, K)], rows_v.at[b], gsem).wait()

    def wait_out(ob):
        pltpu.make_async_copy(out_v.at[ob], sum_hbm.at[pl.ds(0, G)], osem).wait()

    for b in range(NB):
        fire(b, b)

    def reduce_point(b, ob, j):
        def cbody(c, carry):
            col = c * 16
            vals = [rows_v[b, k, pl.ds(col, 16)] for k in range(K)]
            while len(vals) > 1:
                vals = [vals[i] + vals[i + 1] for i in range(0, len(vals), 2)]
            out_v[ob, j, pl.ds(col, 16)] = vals[0]
            return carry

        lax.fori_loop(0, INC // 16, cbody, 0)

    def pair_body(m, carry):
        @pl.when(m < ng_half)
        def _():
            for ob in range(2):
                g = m * 2 + ob

                @pl.when(m > 0)
                def _():
                    wait_out(ob)

                for j in range(G):
                    b = j % NB
                    p = g * G + j
                    wait_gather(b)
                    reduce_point(b, ob, j)
                    nxt = p + NB

                    @pl.when(nxt < ppw)
                    def _():
                        fire(nxt, b)

                pltpu.async_copy(out_v.at[ob],
                                 sum_hbm.at[pl.ds(base + g * G, G)], osem)
        return carry

    lax.fori_loop(0, NGH0, pair_body, 0)

    wait_out(0)
    wait_out(1)


_BR = 1000


def _mlp_body(f_ref, s_ref, w1_ref, b1_ref, w2_ref, b2_ref, o_ref):
    avg = s_ref[...] * (1.0 / K)
    h = jnp.maximum(
        jnp.dot(avg, w1_ref[...], preferred_element_type=jnp.float32)
        + b1_ref[...], 0.0)
    logits = (jnp.dot(h, w2_ref[...], preferred_element_type=jnp.float32)
              + b2_ref[...])
    o_ref[...] = f_ref[...] * jax.nn.sigmoid(logits)


def _mlp(F, sums, W1, b1, W2, b2):
    grid = (N_PTS // _BR,)
    return pl.pallas_call(
        _mlp_body,
        grid=grid,
        in_specs=[
            pl.BlockSpec((_BR, INC), lambda i: (i, 0)),
            pl.BlockSpec((_BR, INC), lambda i: (i, 0)),
            pl.BlockSpec((INC, HIDDEN), lambda i: (0, 0)),
            pl.BlockSpec((1, HIDDEN), lambda i: (0, 0)),
            pl.BlockSpec((HIDDEN, INC), lambda i: (0, 0)),
            pl.BlockSpec((1, INC), lambda i: (0, 0)),
        ],
        out_specs=pl.BlockSpec((_BR, INC), lambda i: (i, 0)),
        out_shape=jax.ShapeDtypeStruct((N_PTS, INC), jnp.float32),
        compiler_params=pltpu.CompilerParams(
            dimension_semantics=("arbitrary",)),
    )(F, sums, W1, b1, W2, b2)


def kernel(F, idx, W1, b1, W2, b2):
    idx32 = idx.astype(jnp.int32)
    idx_pad = jnp.pad(idx32, ((0, PAD_N_IDX - N_PTS), (0, 0)))
    sums = _gather_sum(F, idx_pad.reshape(PAD_N_IDX * K))
    return _mlp(F, sums, W1, b1.reshape(1, HIDDEN), W2, b2.reshape(1, INC))
